# baseline (speedup 1.0000x reference)
import jax
import jax.numpy as jnp
from jax.experimental import pallas as pl
from jax.experimental.pallas import tpu as pltpu

END_TOK = 49407
FILL_ROWS = 1024


def _body(plen_ref, vt_ref, emb_ref, out_ref, end_buf, bnd_buf, sem_end,
          sem_copy, sem_fill, sem_bnd):
    B, L, D = vt_ref.shape
    Lp1 = L + 1

    end_dma = pltpu.make_async_copy(
        emb_ref.at[pl.ds(END_TOK - 7, 8)], end_buf.at[pl.ds(0, 8)], sem_end)
    end_dma.start()
    end_dma.wait()
    end_row = end_buf[7:8, :]
    end_buf[...] = jnp.broadcast_to(end_row, (FILL_ROWS, D))

    def copy_chunks(b, f8, do_start):
        for k in range(11, 2, -1):
            size = 1 << k
            off = pl.multiple_of((f8 >> (k + 1)) << (k + 1), size * 2)
            @pl.when((f8 & size) != 0)
            def _():
                dma = pltpu.make_async_copy(
                    vt_ref.at[b, pl.ds(off, size)],
                    out_ref.at[b, pl.ds(off, size)],
                    sem_copy,
                )
                dma.start() if do_start else dma.wait()

    def fill_chunks(b, f8, do_start):
        s = f8 + 8
        q = jnp.maximum(L - s, 0) >> 3
        for k in range(7, -1, -1):
            rows = 8 << k
            off = pl.multiple_of(s + ((q >> (k + 1)) << (k + 1)) * 8, 8)
            @pl.when((q & (1 << k)) != 0)
            def _():
                dma = pltpu.make_async_copy(
                    end_buf.at[pl.ds(0, rows)],
                    out_ref.at[b, pl.ds(off, rows)],
                    sem_fill,
                )
                dma.start() if do_start else dma.wait()
        dma = pltpu.make_async_copy(
            end_buf.at[pl.ds(0, 1)], out_ref.at[b, pl.ds(L, 1)], sem_fill)
        dma.start() if do_start else dma.wait()

    plens = [plen_ref[b] for b in range(B)]
    f8s = [pl.multiple_of(p & ~7, 8) for p in plens]

    for b in range(B):
        @pl.when(f8s[b] < L)
        def _():
            pltpu.make_async_copy(
                vt_ref.at[b, pl.ds(f8s[b], 8)], bnd_buf.at[b], sem_bnd
            ).start()
    for b in range(B):
        copy_chunks(b, f8s[b], True)
        fill_chunks(b, f8s[b], True)
    rows8 = jax.lax.broadcasted_iota(jnp.int32, (8, 1), 0)
    for b in range(B):
        @pl.when(f8s[b] < L)
        def _():
            pltpu.make_async_copy(
                vt_ref.at[b, pl.ds(f8s[b], 8)], bnd_buf.at[b], sem_bnd
            ).wait()
            r = plens[b] - f8s[b]
            bnd_buf[b] = jnp.where(rows8 < r, bnd_buf[b], end_row)
            pltpu.make_async_copy(
                bnd_buf.at[b], out_ref.at[b, pl.ds(f8s[b], 8)], sem_bnd
            ).start()
    for b in range(B):
        copy_chunks(b, f8s[b], False)
        fill_chunks(b, f8s[b], False)
    for b in range(B):
        @pl.when(f8s[b] < L)
        def _():
            pltpu.make_async_copy(
                bnd_buf.at[b], out_ref.at[b, pl.ds(f8s[b], 8)], sem_bnd
            ).wait()


def kernel(categories, vt, emb):
    B, L = categories.shape
    D = vt.shape[-1]
    plen = jnp.sum((categories != END_TOK).astype(jnp.int32), axis=1)

    grid_spec = pltpu.PrefetchScalarGridSpec(
        num_scalar_prefetch=1,
        grid=(1,),
        in_specs=[
            pl.BlockSpec(memory_space=pl.ANY),
            pl.BlockSpec(memory_space=pl.ANY),
        ],
        out_specs=pl.BlockSpec(memory_space=pl.ANY),
        scratch_shapes=[
            pltpu.VMEM((FILL_ROWS, D), jnp.float32),
            pltpu.VMEM((B, 8, D), jnp.float32),
            pltpu.SemaphoreType.DMA,
            pltpu.SemaphoreType.DMA,
            pltpu.SemaphoreType.DMA,
            pltpu.SemaphoreType.DMA,
        ],
    )

    return pl.pallas_call(
        _body,
        grid_spec=grid_spec,
        out_shape=jax.ShapeDtypeStruct((B, L + 1, D), vt.dtype),
    )(plen, vt, emb)


# speedup vs baseline: 15.5333x; 15.5333x over previous
import jax
import jax.numpy as jnp
from jax.experimental import pallas as pl
from jax.experimental.pallas import tpu as pltpu

END_TOK = 49407


def _body(plen_ref, vt_ref, end_ref, out_ref, sem_copy):
    B, L, D = vt_ref.shape
    b = pl.program_id(0)
    plen = plen_ref[b]
    c8 = pl.multiple_of(jnp.minimum((plen + 7) & ~7, L), 8)

    def copy_chunks(do_start):
        for k in range(11, 2, -1):
            size = 1 << k
            off = pl.multiple_of((c8 >> (k + 1)) << (k + 1), size * 2)
            @pl.when((c8 & size) != 0)
            def _():
                dma = pltpu.make_async_copy(
                    vt_ref.at[b, pl.ds(off, size)],
                    out_ref.at[0, pl.ds(off, size)],
                    sem_copy,
                )
                dma.start() if do_start else dma.wait()

    copy_chunks(True)

    end_row = end_ref[END_TOK % 8:END_TOK % 8 + 1, :]

    q = (L - c8) >> 3
    for k in range(8, -1, -1):
        rows = 8 << k
        off = pl.multiple_of(c8 + ((q >> (k + 1)) << (k + 1)) * 8, 8)
        @pl.when((q & (1 << k)) != 0)
        def _():
            out_ref[0, pl.ds(off, rows)] = jnp.broadcast_to(
                end_row, (rows, D))
    out_ref[0, pl.ds(L, 1)] = end_row

    copy_chunks(False)

    @pl.when(c8 > plen)
    def _():
        f8 = pl.multiple_of(c8 - 8, 8)
        rows8 = jax.lax.broadcasted_iota(jnp.int32, (8, 1), 0) + f8
        tile = out_ref[0, pl.ds(f8, 8)]
        out_ref[0, pl.ds(f8, 8)] = jnp.where(rows8 < plen, tile, end_row)


def kernel(categories, vt, emb):
    B, L = categories.shape
    D = vt.shape[-1]
    plen = jnp.sum((categories != END_TOK).astype(jnp.int32), axis=1)

    grid_spec = pltpu.PrefetchScalarGridSpec(
        num_scalar_prefetch=1,
        grid=(B,),
        in_specs=[
            pl.BlockSpec(memory_space=pl.ANY),
            pl.BlockSpec((8, D), lambda b, p: (END_TOK // 8, 0)),
        ],
        out_specs=pl.BlockSpec((1, L + 1, D), lambda b, p: (b, 0, 0)),
        scratch_shapes=[
            pltpu.SemaphoreType.DMA,
        ],
    )

    return pl.pallas_call(
        _body,
        grid_spec=grid_spec,
        out_shape=jax.ShapeDtypeStruct((B, L + 1, D), vt.dtype),
    )(plen, vt, emb)


# speedup vs baseline: 16.7531x; 1.0785x over previous
import jax
import jax.numpy as jnp
from jax.experimental import pallas as pl
from jax.experimental.pallas import tpu as pltpu

END_TOK = 49407


def _chunk_sizes():
    return [1 << k for k in range(11, 2, -1)]


def _prefix_dma(vt_ref, buf_ref, sem, b, c8, do_start):
    for k in range(11, 2, -1):
        size = 1 << k
        off = pl.multiple_of((c8 >> (k + 1)) << (k + 1), size * 2)
        @pl.when((c8 & size) != 0)
        def _():
            dma = pltpu.make_async_copy(
                vt_ref.at[b, pl.ds(off, size)],
                buf_ref.at[pl.ds(off, size)],
                sem,
            )
            dma.start() if do_start else dma.wait()


def _body(plen_ref, vt_ref, end_ref, out_ref, buf0, buf1, sem0, sem1):
    B, L, D = vt_ref.shape
    b = pl.program_id(0)

    def c8_of(i):
        return pl.multiple_of(jnp.minimum((plen_ref[i] + 7) & ~7, L), 8)

    def stage(cur_buf, cur_sem, nxt_buf, nxt_sem):
        @pl.when(b == 0)
        def _():
            _prefix_dma(vt_ref, cur_buf, cur_sem, 0, c8_of(0), True)

        @pl.when(b + 1 < B)
        def _():
            nxt = b + 1
            _prefix_dma(vt_ref, nxt_buf, nxt_sem, nxt, c8_of(nxt), True)

        _prefix_dma(vt_ref, cur_buf, cur_sem, b, c8_of(b), False)

    plen = plen_ref[b]
    c8 = c8_of(b)
    end_row = end_ref[END_TOK % 8:END_TOK % 8 + 1, :]

    def assemble(buf):
        for k in range(11, 2, -1):
            size = 1 << k
            off = pl.multiple_of((c8 >> (k + 1)) << (k + 1), size * 2)
            @pl.when((c8 & size) != 0)
            def _():
                out_ref[0, pl.ds(off, size)] = buf[pl.ds(off, size)]

        @pl.when(c8 > plen)
        def _():
            f8 = pl.multiple_of(c8 - 8, 8)
            rows8 = jax.lax.broadcasted_iota(jnp.int32, (8, 1), 0) + f8
            tile = buf[pl.ds(f8, 8)]
            out_ref[0, pl.ds(f8, 8)] = jnp.where(rows8 < plen, tile,
                                                 end_row)

    @pl.when(b % 2 == 0)
    def _():
        stage(buf0, sem0, buf1, sem1)
        assemble(buf0)

    @pl.when(b % 2 == 1)
    def _():
        stage(buf1, sem1, buf0, sem0)
        assemble(buf1)

    q = (L - c8) >> 3
    for k in range(8, -1, -1):
        rows = 8 << k
        off = pl.multiple_of(c8 + ((q >> (k + 1)) << (k + 1)) * 8, 8)
        @pl.when((q & (1 << k)) != 0)
        def _():
            out_ref[0, pl.ds(off, rows)] = jnp.broadcast_to(
                end_row, (rows, D))
    out_ref[0, pl.ds(L, 1)] = end_row


def kernel(categories, vt, emb):
    B, L = categories.shape
    D = vt.shape[-1]
    plen = jnp.sum((categories != END_TOK).astype(jnp.int32), axis=1)

    grid_spec = pltpu.PrefetchScalarGridSpec(
        num_scalar_prefetch=1,
        grid=(B,),
        in_specs=[
            pl.BlockSpec(memory_space=pl.ANY),
            pl.BlockSpec((8, D), lambda b, p: (END_TOK // 8, 0)),
        ],
        out_specs=pl.BlockSpec((1, L + 1, D), lambda b, p: (b, 0, 0)),
        scratch_shapes=[
            pltpu.VMEM((L, D), jnp.float32),
            pltpu.VMEM((L, D), jnp.float32),
            pltpu.SemaphoreType.DMA,
            pltpu.SemaphoreType.DMA,
        ],
    )

    return pl.pallas_call(
        _body,
        grid_spec=grid_spec,
        out_shape=jax.ShapeDtypeStruct((B, L + 1, D), vt.dtype),
    )(plen, vt, emb)
